# baseline (speedup 1.0000x reference)
import functools

import jax
import jax.numpy as jnp
from jax.experimental import pallas as pl
from jax.experimental.pallas import tpu as pltpu

TOKENS = 16384
HIDDEN = 4096
EXPERTS = 64
TOP_K = 8
BLOCK = 1024
NBLK = TOKENS // BLOCK


def _router_body(x_ref, w_ref, tw_ref, ti_ref, bl_ref, var_ref, gini_ref,
                 ent_ref, usage_acc, ent_acc):
    i = pl.program_id(0)

    logits = jax.lax.dot_general(
        w_ref[...], x_ref[...],
        dimension_numbers=(((1,), (1,)), ((), ())),
        preferred_element_type=jnp.float32)

    m = jnp.max(logits, axis=0, keepdims=True)
    e = jnp.exp(logits - m)
    s = jnp.sum(e, axis=0, keepdims=True)
    p = e / s

    @pl.when(i == 0)
    def _init():
        usage_acc[...] = jnp.zeros_like(usage_acc)
        ent_acc[...] = jnp.zeros_like(ent_acc)

    usage_acc[...] += jnp.sum(p, axis=1, keepdims=True)
    plogp = p * jnp.log(jnp.clip(p, 1e-9))
    ent_acc[...] += jnp.sum(plogp, keepdims=True).reshape(1, 1)

    iota_e = jax.lax.broadcasted_iota(jnp.int32, (EXPERTS, BLOCK), 0)
    work = p
    tws = []
    tis = []
    for _ in range(TOP_K):
        mk = jnp.max(work, axis=0, keepdims=True)
        hit = work == mk
        idx = jnp.min(jnp.where(hit, iota_e, EXPERTS), axis=0,
                      keepdims=True)
        tws.append(mk)
        tis.append(idx)
        work = jnp.where(iota_e == idx, -1.0, work)

    tw = jnp.concatenate(tws, axis=0)
    ti = jnp.concatenate(tis, axis=0)

    ew = jnp.exp(tw - tw[0:1])
    tw_ref[...] = jnp.transpose(ew / jnp.sum(ew, axis=0, keepdims=True))
    ti_ref[...] = jnp.transpose(ti)

    @pl.when(i == NBLK - 1)
    def _finalize():
        usage = usage_acc[...] * (1.0 / TOKENS)
        total = jnp.sum(usage)
        mean = total * (1.0 / EXPERTS)
        var = jnp.sum((usage - mean) ** 2) * (1.0 / (EXPERTS - 1))
        var_ref[...] = jnp.full((1, 1), var)
        bl_ref[...] = jnp.full((1, 1), var * float(EXPERTS))

        u_cols = jnp.broadcast_to(usage, (EXPERTS, EXPERTS))
        diag = (jax.lax.broadcasted_iota(jnp.int32, (EXPERTS, EXPERTS), 0) ==
                jax.lax.broadcasted_iota(jnp.int32, (EXPERTS, EXPERTS), 1))
        u_rows = jnp.sum(jnp.where(diag, u_cols, 0.0), axis=0,
                         keepdims=True)
        pair = jnp.sum(jnp.abs(u_cols - u_rows))
        denom = 2.0 * EXPERTS * jnp.maximum(total, 1e-9)
        gini_ref[...] = jnp.full((1, 1), pair / denom)

        ent_ref[...] = -ent_acc[...] * (1.0 / TOKENS)


@functools.partial(jax.jit, static_argnames=())
def kernel(x, W):
    tw_t, ti_t, bl, var, gini, ent = pl.pallas_call(
        _router_body,
        grid=(NBLK,),
        in_specs=[
            pl.BlockSpec((BLOCK, HIDDEN), lambda i: (i, 0)),
            pl.BlockSpec((EXPERTS, HIDDEN), lambda i: (0, 0)),
        ],
        out_specs=[
            pl.BlockSpec((BLOCK, TOP_K), lambda i: (i, 0)),
            pl.BlockSpec((BLOCK, TOP_K), lambda i: (i, 0)),
            pl.BlockSpec((1, 1), lambda i: (0, 0)),
            pl.BlockSpec((1, 1), lambda i: (0, 0)),
            pl.BlockSpec((1, 1), lambda i: (0, 0)),
            pl.BlockSpec((1, 1), lambda i: (0, 0)),
        ],
        out_shape=[
            jax.ShapeDtypeStruct((TOKENS, TOP_K), jnp.float32),
            jax.ShapeDtypeStruct((TOKENS, TOP_K), jnp.int32),
            jax.ShapeDtypeStruct((1, 1), jnp.float32),
            jax.ShapeDtypeStruct((1, 1), jnp.float32),
            jax.ShapeDtypeStruct((1, 1), jnp.float32),
            jax.ShapeDtypeStruct((1, 1), jnp.float32),
        ],
        scratch_shapes=[
            pltpu.VMEM((EXPERTS, 1), jnp.float32),
            pltpu.VMEM((1, 1), jnp.float32),
        ],
        compiler_params=pltpu.CompilerParams(
            dimension_semantics=("arbitrary",),
        ),
    )(x, W)
    return (tw_t, ti_t, bl.reshape(()), var.reshape(()),
            gini.reshape(()), ent.reshape(()))


# speedup vs baseline: 1.2047x; 1.2047x over previous
import functools

import jax
import jax.numpy as jnp
from jax.experimental import pallas as pl
from jax.experimental.pallas import tpu as pltpu

TOKENS = 16384
HIDDEN = 4096
EXPERTS = 64
TOP_K = 8
HALF = 512
BLOCK = 2 * HALF
NBLK = TOKENS // BLOCK


def _router_body(xa_ref, xb_ref, w_ref, tw_ref, ti_ref, bl_ref, var_ref,
                 gini_ref, ent_ref, usage_acc, ent_acc):
    i = pl.program_id(0)

    la = jax.lax.dot_general(
        w_ref[...], xa_ref[...],
        dimension_numbers=(((1,), (1,)), ((), ())),
        preferred_element_type=jnp.float32)
    lb = jax.lax.dot_general(
        w_ref[...], xb_ref[...],
        dimension_numbers=(((1,), (1,)), ((), ())),
        preferred_element_type=jnp.float32)
    logits = jnp.concatenate([la, lb], axis=1)

    m = jnp.max(logits, axis=0, keepdims=True)
    e = jnp.exp(logits - m)
    s = jnp.sum(e, axis=0, keepdims=True)
    p = e / s

    @pl.when(i == 0)
    def _init():
        usage_acc[...] = jnp.zeros_like(usage_acc)
        ent_acc[...] = jnp.zeros_like(ent_acc)

    usage_acc[...] += jnp.sum(p, axis=1, keepdims=True)
    plogp = p * jnp.log(jnp.clip(p, 1e-9))
    ent_acc[...] += jnp.sum(plogp, keepdims=True).reshape(1, 1)

    iota_e = jax.lax.broadcasted_iota(jnp.int32, (EXPERTS, BLOCK), 0)
    work = p
    tws = []
    tis = []
    for _ in range(TOP_K):
        mk = jnp.max(work, axis=0, keepdims=True)
        hit = work == mk
        idx = jnp.min(jnp.where(hit, iota_e, EXPERTS), axis=0, keepdims=True)
        tws.append(mk)
        tis.append(idx)
        work = jnp.where(iota_e == idx, -1.0, work)

    tw = jnp.concatenate(tws, axis=0)
    ti = jnp.concatenate(tis, axis=0)

    ew = jnp.exp(tw - tw[0:1])
    tw_ref[...] = ew / jnp.sum(ew, axis=0, keepdims=True)
    ti_ref[...] = ti

    @pl.when(i == NBLK - 1)
    def _finalize():
        usage = usage_acc[...] * (1.0 / TOKENS)
        total = jnp.sum(usage)
        mean = total * (1.0 / EXPERTS)
        var = jnp.sum((usage - mean) ** 2) * (1.0 / (EXPERTS - 1))
        var_ref[...] = jnp.full((1, 1), var)
        bl_ref[...] = jnp.full((1, 1), var * float(EXPERTS))
        u_cols = jnp.broadcast_to(usage, (EXPERTS, EXPERTS))
        diag = (jax.lax.broadcasted_iota(jnp.int32, (EXPERTS, EXPERTS), 0) ==
                jax.lax.broadcasted_iota(jnp.int32, (EXPERTS, EXPERTS), 1))
        u_rows = jnp.sum(jnp.where(diag, u_cols, 0.0), axis=0, keepdims=True)
        pair = jnp.sum(jnp.abs(u_cols - u_rows))
        denom = 2.0 * EXPERTS * jnp.maximum(total, 1e-9)
        gini_ref[...] = jnp.full((1, 1), pair / denom)
        ent_ref[...] = -ent_acc[...] * (1.0 / TOKENS)


@functools.partial(jax.jit, static_argnames=())
def kernel(x, W):
    tw_t, ti_t, bl, var, gini, ent = pl.pallas_call(
        _router_body,
        grid=(NBLK,),
        in_specs=[
            pl.BlockSpec((HALF, HIDDEN), lambda i: (2 * i, 0)),
            pl.BlockSpec((HALF, HIDDEN), lambda i: (2 * i + 1, 0)),
            pl.BlockSpec((EXPERTS, HIDDEN), lambda i: (0, 0)),
        ],
        out_specs=[
            pl.BlockSpec((TOP_K, BLOCK), lambda i: (0, i)),
            pl.BlockSpec((TOP_K, BLOCK), lambda i: (0, i)),
            pl.BlockSpec((1, 1), lambda i: (0, 0)),
            pl.BlockSpec((1, 1), lambda i: (0, 0)),
            pl.BlockSpec((1, 1), lambda i: (0, 0)),
            pl.BlockSpec((1, 1), lambda i: (0, 0)),
        ],
        out_shape=[
            jax.ShapeDtypeStruct((TOP_K, TOKENS), jnp.float32),
            jax.ShapeDtypeStruct((TOP_K, TOKENS), jnp.int32),
            jax.ShapeDtypeStruct((1, 1), jnp.float32),
            jax.ShapeDtypeStruct((1, 1), jnp.float32),
            jax.ShapeDtypeStruct((1, 1), jnp.float32),
            jax.ShapeDtypeStruct((1, 1), jnp.float32),
        ],
        scratch_shapes=[
            pltpu.VMEM((EXPERTS, 1), jnp.float32),
            pltpu.VMEM((1, 1), jnp.float32),
        ],
        compiler_params=pltpu.CompilerParams(
            dimension_semantics=("arbitrary",),
        ),
    )(x, x, W)
    return (tw_t.T, ti_t.T, bl.reshape(()), var.reshape(()),
            gini.reshape(()), ent.reshape(()))
